# baseline (speedup 1.0000x reference)
import jax
import jax.numpy as jnp
from jax.experimental import pallas as pl
from jax.experimental.pallas import tpu as pltpu

D_MODEL = 1024
D_SPACE = 64
S = 2048
N_TOTAL = 16384
N_OUT = 20480
TN = 1024
NUM_J = N_OUT // TN

_HTAB = (0, 0, 1, 1, 2, 2, 3, 3, 3, 3, 4, 4, 5, 5, 6, 6, 7, 7, 7, 7)
_NTAB = (0, 1, 0, 1, 2, 3, 8, 9, 10, 11, 4, 5, 4, 5, 6, 7, 12, 13, 14, 15)


def _ln_into(scr, k, t, g_ref, b_ref):
    g = g_ref[:, k * D_SPACE:(k + 1) * D_SPACE]
    b = b_ref[:, k * D_SPACE:(k + 1) * D_SPACE]
    m = jnp.mean(t, axis=-1, keepdims=True)
    v = jnp.mean((t - m) ** 2, axis=-1, keepdims=True)
    scr[k] = (t - m) * jax.lax.rsqrt(v + 1e-5) * g + b


def _body(ntab_ref, x_ref, ca_ref, ck_ref, ne_ref, Wx_ref, bx_ref, Wr_ref,
          br_ref, Wkn_ref, bkn_ref, g_ref, beta_ref, out_ref, h_scr):
    j = pl.program_id(0)

    @pl.when(j == 0)
    def _prologue():
        px = jnp.dot(x_ref[...], Wx_ref[...],
                     preferred_element_type=jnp.float32) + bx_ref[...]
        pr = jnp.dot(ca_ref[...], Wr_ref[...],
                     preferred_element_type=jnp.float32) + br_ref[...]
        pk = jnp.dot(ck_ref[...], Wkn_ref[...],
                     preferred_element_type=jnp.float32) + bkn_ref[...]
        for k in range(4):
            _ln_into(h_scr, k, px[:, k * D_SPACE:(k + 1) * D_SPACE],
                     g_ref, beta_ref)
        for k in range(3):
            _ln_into(h_scr, 4 + k, pr[:, k * D_SPACE:(k + 1) * D_SPACE],
                     g_ref, beta_ref)
        _ln_into(h_scr, 7, pk, g_ref, beta_ref)

    e = ne_ref[...]
    inv = 1.0 / jnp.maximum(
        jnp.sqrt(jnp.sum(e * e, axis=-1, keepdims=True)), 1e-12)
    en = e * inv
    h = h_scr[ntab_ref[1, j]]
    out_ref[...] = jax.lax.dot_general(
        h, en, (((1,), (1,)), ((), ())), preferred_element_type=jnp.float32)


def kernel(x, ctx_attn, ctx_know, neuron_emb, W_feat, b_feat, W_know, b_know,
           W_rQ, b_rQ, W_rK, b_rK, W_rV, b_rV, W_rKn, b_rKn,
           g_fqkQ, beta_fqkQ, g_fqkK, beta_fqkK, g_fv, beta_fv,
           g_fkn, beta_fkn, g_rQ, beta_rQ, g_rK, beta_rK,
           g_rV, beta_rV, g_rKn, beta_rKn):
    B = x.shape[0]
    x2 = x.reshape(B * S, D_MODEL)
    ca = ctx_attn.reshape(B * S, -1)
    ck = ctx_know.reshape(B * S, -1)

    Wx = jnp.concatenate([W_feat, W_know], axis=1)
    bx = jnp.concatenate([b_feat, b_know])[None, :]
    Wr = jnp.concatenate([W_rQ, W_rK, W_rV], axis=1)
    br = jnp.concatenate([b_rQ, b_rK, b_rV])[None, :]
    bkn = b_rKn[None, :]
    g = jnp.concatenate([g_fqkQ, g_fqkK, g_fv, g_fkn,
                         g_rQ, g_rK, g_rV, g_rKn])[None, :]
    beta = jnp.concatenate([beta_fqkQ, beta_fqkK, beta_fv, beta_fkn,
                            beta_rQ, beta_rK, beta_rV, beta_rKn])[None, :]

    ntab = jnp.asarray([_NTAB, _HTAB], dtype=jnp.int32)
    full = lambda a: pl.BlockSpec(a.shape, lambda j, tab: (0,) * a.ndim)

    grid_spec = pltpu.PrefetchScalarGridSpec(
        num_scalar_prefetch=1,
        grid=(NUM_J,),
        in_specs=[
            full(x2), full(ca), full(ck),
            pl.BlockSpec((TN, D_SPACE), lambda j, tab: (tab[0, j], 0)),
            full(Wx), full(bx), full(Wr), full(br),
            pl.BlockSpec(W_rKn.shape, lambda j, tab: (0, 0)), full(bkn),
            full(g), full(beta),
        ],
        out_specs=pl.BlockSpec((B * S, TN), lambda j, tab: (0, j)),
        scratch_shapes=[pltpu.VMEM((8, B * S, D_SPACE), jnp.float32)],
    )

    out = pl.pallas_call(
        _body,
        grid_spec=grid_spec,
        out_shape=jax.ShapeDtypeStruct((B * S, N_OUT), jnp.float32),
    )(ntab, x2, ca, ck, neuron_emb, Wx, bx, Wr, br, W_rKn, bkn, g, beta)

    return out.reshape(B, S, N_OUT)


# speedup vs baseline: 1.0021x; 1.0021x over previous
import jax
import jax.numpy as jnp
from jax.experimental import pallas as pl
from jax.experimental.pallas import tpu as pltpu

D_MODEL = 1024
D_SPACE = 64
S = 2048
N_TOTAL = 16384
N_OUT = 20480
TN = 1024
NUM_J = N_OUT // TN

_HTAB = (0, 0, 1, 1, 2, 2, 3, 3, 3, 3, 4, 4, 5, 5, 6, 6, 7, 7, 7, 7)
_NTAB = (0, 1, 0, 1, 2, 3, 8, 9, 10, 11, 4, 5, 4, 5, 6, 7, 12, 13, 14, 15)


def _ln_into(scr, k, t, g_ref, b_ref):
    g = g_ref[:, k * D_SPACE:(k + 1) * D_SPACE]
    b = b_ref[:, k * D_SPACE:(k + 1) * D_SPACE]
    m = jnp.mean(t, axis=-1, keepdims=True)
    v = jnp.mean((t - m) ** 2, axis=-1, keepdims=True)
    scr[k] = ((t - m) * jax.lax.rsqrt(v + 1e-5) * g + b).astype(jnp.bfloat16)


def _body(ntab_ref, x_ref, ca_ref, ck_ref, ne_ref, Wx_ref, bx_ref, Wr_ref,
          br_ref, Wkn_ref, bkn_ref, g_ref, beta_ref, out_ref, h_scr):
    j = pl.program_id(0)

    @pl.when(j == 0)
    def _prologue():
        px = jnp.dot(x_ref[...], Wx_ref[...],
                     preferred_element_type=jnp.float32) + bx_ref[...]
        pr = jnp.dot(ca_ref[...], Wr_ref[...],
                     preferred_element_type=jnp.float32) + br_ref[...]
        pk = jnp.dot(ck_ref[...], Wkn_ref[...],
                     preferred_element_type=jnp.float32) + bkn_ref[...]
        for k in range(4):
            _ln_into(h_scr, k, px[:, k * D_SPACE:(k + 1) * D_SPACE],
                     g_ref, beta_ref)
        for k in range(3):
            _ln_into(h_scr, 4 + k, pr[:, k * D_SPACE:(k + 1) * D_SPACE],
                     g_ref, beta_ref)
        _ln_into(h_scr, 7, pk, g_ref, beta_ref)

    e = ne_ref[...]
    inv = 1.0 / jnp.maximum(
        jnp.sqrt(jnp.sum(e * e, axis=-1, keepdims=True)), 1e-12)
    en = (e * inv).astype(jnp.bfloat16)
    h = h_scr[ntab_ref[1, j]]
    out_ref[...] = jax.lax.dot_general(
        h, en, (((1,), (1,)), ((), ())), preferred_element_type=jnp.float32)


def kernel(x, ctx_attn, ctx_know, neuron_emb, W_feat, b_feat, W_know, b_know,
           W_rQ, b_rQ, W_rK, b_rK, W_rV, b_rV, W_rKn, b_rKn,
           g_fqkQ, beta_fqkQ, g_fqkK, beta_fqkK, g_fv, beta_fv,
           g_fkn, beta_fkn, g_rQ, beta_rQ, g_rK, beta_rK,
           g_rV, beta_rV, g_rKn, beta_rKn):
    B = x.shape[0]
    x2 = x.reshape(B * S, D_MODEL)
    ca = ctx_attn.reshape(B * S, -1)
    ck = ctx_know.reshape(B * S, -1)

    Wx = jnp.concatenate([W_feat, W_know], axis=1)
    bx = jnp.concatenate([b_feat, b_know])[None, :]
    Wr = jnp.concatenate([W_rQ, W_rK, W_rV], axis=1)
    br = jnp.concatenate([b_rQ, b_rK, b_rV])[None, :]
    bkn = b_rKn[None, :]
    g = jnp.concatenate([g_fqkQ, g_fqkK, g_fv, g_fkn,
                         g_rQ, g_rK, g_rV, g_rKn])[None, :]
    beta = jnp.concatenate([beta_fqkQ, beta_fqkK, beta_fv, beta_fkn,
                            beta_rQ, beta_rK, beta_rV, beta_rKn])[None, :]

    ntab = jnp.asarray([_NTAB, _HTAB], dtype=jnp.int32)
    full = lambda a: pl.BlockSpec(a.shape, lambda j, tab: (0,) * a.ndim)

    grid_spec = pltpu.PrefetchScalarGridSpec(
        num_scalar_prefetch=1,
        grid=(NUM_J,),
        in_specs=[
            full(x2), full(ca), full(ck),
            pl.BlockSpec((TN, D_SPACE), lambda j, tab: (tab[0, j], 0)),
            full(Wx), full(bx), full(Wr), full(br),
            pl.BlockSpec(W_rKn.shape, lambda j, tab: (0, 0)), full(bkn),
            full(g), full(beta),
        ],
        out_specs=pl.BlockSpec((B * S, TN), lambda j, tab: (0, j)),
        scratch_shapes=[pltpu.VMEM((8, B * S, D_SPACE), jnp.bfloat16)],
    )

    out = pl.pallas_call(
        _body,
        grid_spec=grid_spec,
        out_shape=jax.ShapeDtypeStruct((B * S, N_OUT), jnp.float32),
    )(ntab, x2, ca, ck, neuron_emb, Wx, bx, Wr, br, W_rKn, bkn, g, beta)

    return out.reshape(B, S, N_OUT)
